# Initial kernel scaffold; baseline (speedup 1.0000x reference)
import jax
import jax.numpy as jnp
from jax.experimental import pallas as pl


def kernel(features, edge_index, W0, b0, W1, b1, W2, b2):
    raise NotImplementedError("write your pallas kernel here")


# speedup vs baseline: 6.9654x; 6.9654x over previous
import functools

import jax
import jax.numpy as jnp
from jax import lax
from jax.experimental import pallas as pl
from jax.experimental.pallas import tpu as pltpu
from jax.experimental.pallas import tpu_sc as plsc

N_NODES = 10000
N_EDGES = 320000
F = 128

NC = 2
NS = 16
NW = NC * NS
EPW = N_EDGES // NW
CHUNK = 80
NCHUNK = EPW // CHUNK
ROWS_PT = 624
TAIL = N_NODES - NS * ROWS_PT

_sc_mesh = plsc.VectorSubcoreMesh(core_axis_name="c", subcore_axis_name="s")


@functools.partial(
    pl.kernel,
    out_type=jax.ShapeDtypeStruct((NC, N_NODES, F), jnp.float32),
    mesh=_sc_mesh,
    scratch_types=[
        pltpu.VMEM((NCHUNK, CHUNK), jnp.int32),
        pltpu.VMEM((NCHUNK, CHUNK), jnp.int32),
        pltpu.VMEM((CHUNK, F), jnp.float32),
        pltpu.VMEM_SHARED((N_NODES, F), jnp.float32),
        pltpu.SemaphoreType.DMA,
    ],
)
def _sc_aggregate(y_hbm, srcs_hbm, dsts_hbm, zeros_hbm, out_hbm,
                  src_v, dst_v, buf, acc, sem):
    c = lax.axis_index("c")
    s = lax.axis_index("s")
    wid = c * NS + s
    pltpu.sync_copy(zeros_hbm, acc.at[pl.ds(s * ROWS_PT, ROWS_PT)])

    @pl.when(s == NS - 1)
    def _():
        pltpu.sync_copy(zeros_hbm.at[pl.ds(0, TAIL)],
                        acc.at[pl.ds(NS * ROWS_PT, TAIL)])
    pltpu.sync_copy(srcs_hbm.at[wid], src_v)
    pltpu.sync_copy(dsts_hbm.at[wid], dst_v)
    plsc.subcore_barrier()

    def chunk_body(i, carry):
        pltpu.async_copy(y_hbm.at[src_v.at[i]], buf, sem).wait()
        pltpu.sync_copy(buf, acc.at[dst_v.at[i]], add=True)
        return carry

    lax.fori_loop(0, NCHUNK, chunk_body, 0)
    plsc.subcore_barrier()
    pltpu.sync_copy(acc.at[pl.ds(s * ROWS_PT, ROWS_PT)],
                    out_hbm.at[c, pl.ds(s * ROWS_PT, ROWS_PT)])

    @pl.when(s == NS - 1)
    def _():
        pltpu.sync_copy(acc.at[pl.ds(NS * ROWS_PT, TAIL)],
                        out_hbm.at[c, pl.ds(NS * ROWS_PT, TAIL)])


_BLK = 1000


def _mm_first_body(x_ref, w_ref, o_ref):
    o_ref[...] = lax.dot_general(
        x_ref[...], w_ref[...], (((1,), (1,)), ((), ())),
        preferred_element_type=jnp.float32)


def _mm_fused_body(p_ref, q_ref, b_ref, w_ref, o_ref):
    h = p_ref[...] + q_ref[...] + b_ref[...]
    o_ref[...] = lax.dot_general(
        h, w_ref[...], (((1,), (1,)), ((), ())),
        preferred_element_type=jnp.float32)


def _add_bias_body(p_ref, q_ref, b_ref, o_ref):
    o_ref[...] = p_ref[...] + q_ref[...] + b_ref[...]


_row_spec = pl.BlockSpec((_BLK, F), lambda i: (i, 0))
_b_spec = pl.BlockSpec((1, F), lambda i: (0, 0))
_w_spec = pl.BlockSpec((F, F), lambda i: (0, 0))
_out_shape = jax.ShapeDtypeStruct((N_NODES, F), jnp.float32)

_mm_first = pl.pallas_call(
    _mm_first_body, grid=(N_NODES // _BLK,),
    in_specs=[_row_spec, _w_spec], out_specs=_row_spec,
    out_shape=_out_shape)

_mm_fused = pl.pallas_call(
    _mm_fused_body, grid=(N_NODES // _BLK,),
    in_specs=[_row_spec, _row_spec, _b_spec, _w_spec], out_specs=_row_spec,
    out_shape=_out_shape)

_add_bias = pl.pallas_call(
    _add_bias_body, grid=(N_NODES // _BLK,),
    in_specs=[_row_spec, _row_spec, _b_spec], out_specs=_row_spec,
    out_shape=_out_shape)


def kernel(features, edge_index, W0, b0, W1, b1, W2, b2):
    src = edge_index[0].astype(jnp.int32).reshape(NW, NCHUNK, CHUNK)
    dst = edge_index[1].astype(jnp.int32).reshape(NW, NCHUNK, CHUNK)
    zeros = jnp.zeros((ROWS_PT, F), jnp.float32)

    y = _mm_first(features, W0)
    p = _sc_aggregate(y, src, dst, zeros)
    y = _mm_fused(p[0], p[1], b0.reshape(1, F), W1)
    p = _sc_aggregate(y, src, dst, zeros)
    y = _mm_fused(p[0], p[1], b1.reshape(1, F), W2)
    p = _sc_aggregate(y, src, dst, zeros)
    return _add_bias(p[0], p[1], b2.reshape(1, F))
